# Initial kernel scaffold; baseline (speedup 1.0000x reference)
import jax
import jax.numpy as jnp
from jax.experimental import pallas as pl


def kernel(x, inc_w1, inc_s1, inc_w2, inc_s2, down1_w1, down1_s1, down1_w2, down1_s2, down2_w1, down2_s1, down2_w2, down2_s2, down3_w1, down3_s1, down3_w2, down3_s2, down4_w1, down4_s1, down4_w2, down4_s2, up1_w1, up1_s1, up1_w2, up1_s2, up2_w1, up2_s1, up2_w2, up2_s2, up3_w1, up3_s1, up3_w2, up3_s2, up4_w1, up4_s1, up4_w2, up4_s2, outc_w, outc_s):
    raise NotImplementedError("write your pallas kernel here")


# speedup vs baseline: 2.7105x; 2.7105x over previous
import functools

import jax
import jax.numpy as jnp
from jax.experimental import pallas as pl
from jax.experimental.pallas import tpu as pltpu

_SLOPE = 0.01
_VMEM_LIMIT = 50 * 1024 * 1024
N_CLASSES = 19


def _conv_body(*args, nin, cins, th, w, pack, fuse):
    xs = args[0:nin]
    ws = args[nin:2 * nin]
    sref = args[2 * nin]
    if fuse:
        owr, osr, oref = args[2 * nin + 1], args[2 * nin + 2], args[2 * nin + 3]
    else:
        oref = args[2 * nin + 1]
    r0 = pl.program_id(1) * th
    cout = ws[0].shape[1]
    rows = th * w

    acc = jnp.zeros((rows, cout), jnp.float32)
    for xr, wr, c in zip(xs, ws, cins):
        if pack:
            for dy in range(3):
                slab = jnp.concatenate(
                    [xr[0, pl.ds(r0 + dy, th), pl.ds(dx, w), :] for dx in range(3)],
                    axis=-1).reshape(rows, 3 * c)
                acc += jnp.dot(slab, wr[dy * 3 * c:(dy + 1) * 3 * c, :],
                               preferred_element_type=jnp.float32)
        else:
            for dy in range(3):
                for dx in range(3):
                    xt = xr[0, pl.ds(r0 + dy, th), pl.ds(dx, w), :].reshape(rows, c)
                    t = (dy * 3 + dx) * c
                    acc += jnp.dot(xt, wr[t:t + c, :],
                                   preferred_element_type=jnp.float32)
    y = acc + sref[...]
    y = jnp.where(y >= 0.0, y, _SLOPE * y)
    if fuse:
        z = jnp.dot(y.astype(jnp.bfloat16), owr[...],
                    preferred_element_type=jnp.float32) + osr[...]
        oref[0] = z.reshape(th, w, osr.shape[-1]).astype(oref.dtype)
    else:
        oref[0] = y.reshape(th, w, cout).astype(oref.dtype)


def _halo_chunks(x, nch):
    n, h, w, c = x.shape
    xp = jnp.pad(x, ((0, 0), (1, 1), (1, 1), (0, 0)))
    if nch == 1:
        return xp
    hc = h // nch
    bands = jnp.stack([xp[:, i * hc:i * hc + hc + 2] for i in range(nch)], axis=1)
    return bands.reshape(n * nch, hc + 2, w + 2, c)


def _conv3x3(xs_raw, ws, shift, *, fuse_1x1=None, out_dtype=jnp.bfloat16):
    n0, h0, w, _ = xs_raw[0].shape
    cins = [xi.shape[-1] for xi in xs_raw]
    cmax = max(cins)
    cout = ws[0].shape[1]
    pack = cmax <= 128
    win_bytes = sum((h0 + 2) * (w + 2) * c * 2 for c in cins)
    nch = 1
    while win_bytes // nch > 6 * 1024 * 1024 and h0 // nch >= 16:
        nch *= 2
    xs = [_halo_chunks(xi, nch) for xi in xs_raw]
    n, hp, wp, _ = xs[0].shape
    h = hp - 2
    rows_t = 2048 if cmax <= 128 else (1024 if cmax <= 256 else 512)
    th = min(h, max(1, rows_t // w))
    num_h = h // th
    nin = len(xs)
    fuse = fuse_1x1 is not None

    in_specs = [pl.BlockSpec((1, hp, wp, xi.shape[-1]), lambda ni, hi: (ni, 0, 0, 0))
                for xi in xs]
    in_specs += [pl.BlockSpec(wi.shape, lambda ni, hi: (0, 0)) for wi in ws]
    in_specs.append(pl.BlockSpec(shift.shape, lambda ni, hi: (0, 0)))
    args = list(xs) + list(ws) + [shift]
    if fuse:
        ow, osv = fuse_1x1
        in_specs += [pl.BlockSpec(ow.shape, lambda ni, hi: (0, 0)),
                     pl.BlockSpec(osv.shape, lambda ni, hi: (0, 0))]
        args += [ow, osv]
        c_final = ow.shape[1]
    else:
        c_final = cout

    body = functools.partial(_conv_body, nin=nin, cins=cins, th=th, w=w,
                             pack=pack, fuse=fuse)
    out = pl.pallas_call(
        body,
        out_shape=jax.ShapeDtypeStruct((n, h, w, c_final), out_dtype),
        grid_spec=pltpu.PrefetchScalarGridSpec(
            num_scalar_prefetch=0,
            grid=(n, num_h),
            in_specs=in_specs,
            out_specs=pl.BlockSpec((1, th, w, c_final),
                                   lambda ni, hi: (ni, hi, 0, 0)),
        ),
        compiler_params=pltpu.CompilerParams(
            dimension_semantics=("parallel", "parallel"),
            vmem_limit_bytes=_VMEM_LIMIT,
        ),
    )(*args)
    return out.reshape(n0, h0, w, out.shape[-1])


def _mm_body(x_ref, w_ref, s_ref, o_ref):
    y = jnp.dot(x_ref[...], w_ref[...],
                preferred_element_type=jnp.float32) + s_ref[...]
    y = jnp.where(y >= 0.0, y, _SLOPE * y)
    o_ref[...] = y.astype(o_ref.dtype)


def _entry_conv(x, w2d, shift):
    n, h, w, c = x.shape
    m = n * h * w
    cout = w2d.shape[1]
    xp = jnp.pad(x, ((0, 0), (1, 1), (1, 1), (0, 0)))
    taps = [xp[:, dy:dy + h, dx:dx + w, :] for dy in range(3) for dx in range(3)]
    pat = jnp.stack(taps, axis=3).reshape(m, 9 * c)
    tm = min(m, 4096)
    y = pl.pallas_call(
        _mm_body,
        out_shape=jax.ShapeDtypeStruct((m, cout), jnp.bfloat16),
        grid_spec=pltpu.PrefetchScalarGridSpec(
            num_scalar_prefetch=0,
            grid=(m // tm,),
            in_specs=[pl.BlockSpec((tm, 9 * c), lambda i: (i, 0)),
                      pl.BlockSpec(w2d.shape, lambda i: (0, 0)),
                      pl.BlockSpec(shift.shape, lambda i: (0, 0))],
            out_specs=pl.BlockSpec((tm, cout), lambda i: (i, 0)),
        ),
        compiler_params=pltpu.CompilerParams(
            dimension_semantics=("parallel",),
            vmem_limit_bytes=_VMEM_LIMIT,
        ),
    )(pat, w2d, shift)
    return y.reshape(n, h, w, cout)


def _pool_body(a, b, c, d, o):
    o[...] = jnp.maximum(jnp.maximum(a[...], b[...]),
                         jnp.maximum(c[...], d[...]))


def _maxpool(x):
    n, h, w, c = x.shape
    h2, w2 = h // 2, w // 2
    m = n * h2 * w2
    parts = [x[:, i::2, j::2, :].reshape(m, c) for i in (0, 1) for j in (0, 1)]
    tm = min(m, 4096)
    y = pl.pallas_call(
        _pool_body,
        out_shape=jax.ShapeDtypeStruct((m, c), x.dtype),
        grid_spec=pltpu.PrefetchScalarGridSpec(
            num_scalar_prefetch=0,
            grid=(m // tm,),
            in_specs=[pl.BlockSpec((tm, c), lambda i: (i, 0))] * 4,
            out_specs=pl.BlockSpec((tm, c), lambda i: (i, 0)),
        ),
        compiler_params=pltpu.CompilerParams(
            dimension_semantics=("parallel",),
            vmem_limit_bytes=_VMEM_LIMIT,
        ),
    )(*parts)
    return y.reshape(n, h2, w2, c)


def _up2(x):
    for axis in (1, 2):
        size = x.shape[axis]
        m = 2 * size
        pos = jnp.arange(m, dtype=jnp.float32) * ((size - 1) / (m - 1))
        lo = jnp.minimum(pos.astype(jnp.int32), size - 2)
        t = pos - lo.astype(jnp.float32)
        shape = [1, 1, 1, 1]
        shape[axis] = m
        t = t.reshape(shape)
        a = jnp.take(x, lo, axis=axis)
        b = jnp.take(x, lo + 1, axis=axis)
        x = a * (1.0 - t) + b * t
    return x.astype(jnp.bfloat16)


def _split_w(w2d, ca, cb):
    cout = w2d.shape[1]
    w9 = w2d.reshape(9, ca + cb, cout)
    return (w9[:, :ca, :].reshape(9 * ca, cout),
            w9[:, ca:, :].reshape(9 * cb, cout))


def kernel(x, inc_w1, inc_s1, inc_w2, inc_s2,
           down1_w1, down1_s1, down1_w2, down1_s2,
           down2_w1, down2_s1, down2_w2, down2_s2,
           down3_w1, down3_s1, down3_w2, down3_s2,
           down4_w1, down4_s1, down4_w2, down4_s2,
           up1_w1, up1_s1, up1_w2, up1_s2,
           up2_w1, up2_s1, up2_w2, up2_s2,
           up3_w1, up3_s1, up3_w2, up3_s2,
           up4_w1, up4_s1, up4_w2, up4_s2,
           outc_w, outc_s):
    xh = jnp.transpose(x, (0, 2, 3, 1)).astype(jnp.bfloat16)

    def dconv(a, w1, s1, w2, s2):
        t = _conv3x3([a], [w1], s1)
        return _conv3x3([t], [w2], s2)

    t = _entry_conv(xh, inc_w1, inc_s1)
    x1 = _conv3x3([t], [inc_w2], inc_s2)
    x2 = dconv(_maxpool(x1), down1_w1, down1_s1, down1_w2, down1_s2)
    x3 = dconv(_maxpool(x2), down2_w1, down2_s1, down2_w2, down2_s2)
    x4 = dconv(_maxpool(x3), down3_w1, down3_s1, down3_w2, down3_s2)
    x5 = dconv(_maxpool(x4), down4_w1, down4_s1, down4_w2, down4_s2)

    def up_in(xlow, skip, w1, s1):
        u = _up2(xlow)
        wa, wb = _split_w(w1, skip.shape[-1], u.shape[-1])
        return _conv3x3([skip, u], [wa, wb], s1)

    y = up_in(x5, x4, up1_w1, up1_s1)
    y = _conv3x3([y], [up1_w2], up1_s2)
    y = up_in(y, x3, up2_w1, up2_s1)
    y = _conv3x3([y], [up2_w2], up2_s2)
    y = up_in(y, x2, up3_w1, up3_s1)
    y = _conv3x3([y], [up3_w2], up3_s2)
    y = up_in(y, x1, up4_w1, up4_s1)

    logits = _conv3x3(
        [y], [up4_w2], up4_s2,
        fuse_1x1=(outc_w[:, :N_CLASSES], outc_s[:, :N_CLASSES]),
        out_dtype=jnp.float32)
    return jnp.transpose(logits, (0, 3, 1, 2))


# speedup vs baseline: 4.5266x; 1.6700x over previous
import functools

import jax
import jax.numpy as jnp
from jax.experimental import pallas as pl
from jax.experimental.pallas import tpu as pltpu

_SLOPE = 0.01
_VMEM_LIMIT = 50 * 1024 * 1024
N_CLASSES = 19


def _conv_body(*args, nin, cins, th, w, pack, fuse, pool):
    xs = args[0:nin]
    ws = args[nin:2 * nin]
    sref = args[2 * nin]
    pref = None
    if fuse:
        owr, osr, oref = args[2 * nin + 1], args[2 * nin + 2], args[2 * nin + 3]
    elif pool:
        oref, pref = args[2 * nin + 1], args[2 * nin + 2]
    else:
        oref = args[2 * nin + 1]
    r0 = pl.program_id(1) * th
    cout = ws[0].shape[1]
    rows = th * w

    acc = jnp.zeros((rows, cout), jnp.float32)
    for xr, wr, c in zip(xs, ws, cins):
        if pack:
            for dy in range(3):
                slab = jnp.concatenate(
                    [xr[0, pl.ds(r0 + dy, th), pl.ds(dx, w), :] for dx in range(3)],
                    axis=-1).reshape(rows, 3 * c)
                acc += jnp.dot(slab, wr[dy * 3 * c:(dy + 1) * 3 * c, :],
                               preferred_element_type=jnp.float32)
        else:
            for dy in range(3):
                for dx in range(3):
                    xt = xr[0, pl.ds(r0 + dy, th), pl.ds(dx, w), :].reshape(rows, c)
                    t = (dy * 3 + dx) * c
                    acc += jnp.dot(xt, wr[t:t + c, :],
                                   preferred_element_type=jnp.float32)
    y = acc + sref[...]
    y = jnp.where(y >= 0.0, y, _SLOPE * y)
    if fuse:
        z = jnp.dot(y.astype(jnp.bfloat16), owr[...],
                    preferred_element_type=jnp.float32) + osr[...]
        oref[0] = z.reshape(th, w, osr.shape[-1]).astype(oref.dtype)
    else:
        yb = y.reshape(th, w, cout).astype(oref.dtype)
        oref[0] = yb
        if pool:
            ph = yb.reshape(th // 2, 2, w, cout).max(axis=1)
            p = ph.reshape(th // 2, w // 2, 2, cout).max(axis=2)
            pref[0] = p


def _halo_chunks(x, nch):
    n, h, w, c = x.shape
    xp = jnp.pad(x, ((0, 0), (1, 1), (1, 1), (0, 0)))
    if nch == 1:
        return xp
    hc = h // nch
    bands = jnp.stack([xp[:, i * hc:i * hc + hc + 2] for i in range(nch)], axis=1)
    return bands.reshape(n * nch, hc + 2, w + 2, c)


def _conv3x3(xs_raw, ws, shift, *, fuse_1x1=None, pool=False,
             out_dtype=jnp.bfloat16):
    n0, h0, w, _ = xs_raw[0].shape
    cins = [xi.shape[-1] for xi in xs_raw]
    cmax = max(cins)
    cout = ws[0].shape[1]
    pack = cmax <= 128
    win_bytes = sum((h0 + 2) * (w + 2) * c * 2 for c in cins)
    nch = 1
    while win_bytes // nch > 6 * 1024 * 1024 and h0 // nch >= 16:
        nch *= 2
    xs = [_halo_chunks(xi, nch) for xi in xs_raw]
    n, hp, wp, _ = xs[0].shape
    h = hp - 2
    rows_t = 2048 if cmax <= 128 else (1024 if cmax <= 256 else 512)
    th = min(h, max(1, rows_t // w))
    num_h = h // th
    nin = len(xs)
    fuse = fuse_1x1 is not None

    in_specs = [pl.BlockSpec((1, hp, wp, xi.shape[-1]), lambda ni, hi: (ni, 0, 0, 0))
                for xi in xs]
    in_specs += [pl.BlockSpec(wi.shape, lambda ni, hi: (0, 0)) for wi in ws]
    in_specs.append(pl.BlockSpec(shift.shape, lambda ni, hi: (0, 0)))
    args = list(xs) + list(ws) + [shift]
    if fuse:
        ow, osv = fuse_1x1
        in_specs += [pl.BlockSpec(ow.shape, lambda ni, hi: (0, 0)),
                     pl.BlockSpec(osv.shape, lambda ni, hi: (0, 0))]
        args += [ow, osv]
        c_final = ow.shape[1]
    else:
        c_final = cout

    body = functools.partial(_conv_body, nin=nin, cins=cins, th=th, w=w,
                             pack=pack, fuse=fuse, pool=pool)
    out_shape = [jax.ShapeDtypeStruct((n, h, w, c_final), out_dtype)]
    out_specs = [pl.BlockSpec((1, th, w, c_final), lambda ni, hi: (ni, hi, 0, 0))]
    if pool:
        out_shape.append(jax.ShapeDtypeStruct((n, h // 2, w // 2, c_final),
                                              out_dtype))
        out_specs.append(pl.BlockSpec((1, th // 2, w // 2, c_final),
                                      lambda ni, hi: (ni, hi, 0, 0)))
    res = pl.pallas_call(
        body,
        out_shape=out_shape,
        grid_spec=pltpu.PrefetchScalarGridSpec(
            num_scalar_prefetch=0,
            grid=(n, num_h),
            in_specs=in_specs,
            out_specs=out_specs,
        ),
        compiler_params=pltpu.CompilerParams(
            dimension_semantics=("parallel", "parallel"),
            vmem_limit_bytes=_VMEM_LIMIT,
        ),
    )(*args)
    out = res[0].reshape(n0, h0, w, res[0].shape[-1])
    if pool:
        return out, res[1].reshape(n0, h0 // 2, w // 2, res[1].shape[-1])
    return out


def _mm_body(x_ref, w_ref, s_ref, o_ref):
    y = jnp.dot(x_ref[...], w_ref[...],
                preferred_element_type=jnp.float32) + s_ref[...]
    y = jnp.where(y >= 0.0, y, _SLOPE * y)
    o_ref[...] = y.astype(o_ref.dtype)


def _entry_conv(x, w2d, shift):
    n, h, w, c = x.shape
    m = n * h * w
    cout = w2d.shape[1]
    xp = jnp.pad(x, ((0, 0), (1, 1), (1, 1), (0, 0)))
    taps = [xp[:, dy:dy + h, dx:dx + w, :] for dy in range(3) for dx in range(3)]
    pat = jnp.stack(taps, axis=3).reshape(m, 9 * c)
    tm = min(m, 4096)
    y = pl.pallas_call(
        _mm_body,
        out_shape=jax.ShapeDtypeStruct((m, cout), jnp.bfloat16),
        grid_spec=pltpu.PrefetchScalarGridSpec(
            num_scalar_prefetch=0,
            grid=(m // tm,),
            in_specs=[pl.BlockSpec((tm, 9 * c), lambda i: (i, 0)),
                      pl.BlockSpec(w2d.shape, lambda i: (0, 0)),
                      pl.BlockSpec(shift.shape, lambda i: (0, 0))],
            out_specs=pl.BlockSpec((tm, cout), lambda i: (i, 0)),
        ),
        compiler_params=pltpu.CompilerParams(
            dimension_semantics=("parallel",),
            vmem_limit_bytes=_VMEM_LIMIT,
        ),
    )(pat, w2d, shift)
    return y.reshape(n, h, w, cout)


def _up2(x):
    for axis in (1, 2):
        size = x.shape[axis]
        m = 2 * size
        pos = jnp.arange(m, dtype=jnp.float32) * ((size - 1) / (m - 1))
        lo = jnp.minimum(pos.astype(jnp.int32), size - 2)
        t = pos - lo.astype(jnp.float32)
        shape = [1, 1, 1, 1]
        shape[axis] = m
        t = t.reshape(shape)
        a = jnp.take(x, lo, axis=axis)
        b = jnp.take(x, lo + 1, axis=axis)
        x = a * (1.0 - t) + b * t
    return x.astype(jnp.bfloat16)


def _split_w(w2d, ca, cb):
    cout = w2d.shape[1]
    w9 = w2d.reshape(9, ca + cb, cout)
    return (w9[:, :ca, :].reshape(9 * ca, cout),
            w9[:, ca:, :].reshape(9 * cb, cout))


def kernel(x, inc_w1, inc_s1, inc_w2, inc_s2,
           down1_w1, down1_s1, down1_w2, down1_s2,
           down2_w1, down2_s1, down2_w2, down2_s2,
           down3_w1, down3_s1, down3_w2, down3_s2,
           down4_w1, down4_s1, down4_w2, down4_s2,
           up1_w1, up1_s1, up1_w2, up1_s2,
           up2_w1, up2_s1, up2_w2, up2_s2,
           up3_w1, up3_s1, up3_w2, up3_s2,
           up4_w1, up4_s1, up4_w2, up4_s2,
           outc_w, outc_s):
    xh = jnp.transpose(x, (0, 2, 3, 1)).astype(jnp.bfloat16)

    t = _entry_conv(xh, inc_w1, inc_s1)
    x1, p = _conv3x3([t], [inc_w2], inc_s2, pool=True)
    t = _conv3x3([p], [down1_w1], down1_s1)
    x2, p = _conv3x3([t], [down1_w2], down1_s2, pool=True)
    t = _conv3x3([p], [down2_w1], down2_s1)
    x3, p = _conv3x3([t], [down2_w2], down2_s2, pool=True)
    t = _conv3x3([p], [down3_w1], down3_s1)
    x4, p = _conv3x3([t], [down3_w2], down3_s2, pool=True)
    t = _conv3x3([p], [down4_w1], down4_s1)
    x5 = _conv3x3([t], [down4_w2], down4_s2)

    def up_in(xlow, skip, w1, s1):
        u = _up2(xlow)
        wa, wb = _split_w(w1, skip.shape[-1], u.shape[-1])
        return _conv3x3([skip, u], [wa, wb], s1)

    y = up_in(x5, x4, up1_w1, up1_s1)
    y = _conv3x3([y], [up1_w2], up1_s2)
    y = up_in(y, x3, up2_w1, up2_s1)
    y = _conv3x3([y], [up2_w2], up2_s2)
    y = up_in(y, x2, up3_w1, up3_s1)
    y = _conv3x3([y], [up3_w2], up3_s2)
    y = up_in(y, x1, up4_w1, up4_s1)

    logits = _conv3x3(
        [y], [up4_w2], up4_s2,
        fuse_1x1=(outc_w[:, :N_CLASSES], outc_s[:, :N_CLASSES]),
        out_dtype=jnp.float32)
    return jnp.transpose(logits, (0, 3, 1, 2))
